# Initial kernel scaffold; baseline (speedup 1.0000x reference)
import jax
import jax.numpy as jnp
from jax.experimental import pallas as pl


def kernel(x_t0, edge_weight_t, x_s0, edge_weight_s, Wt0, Wt1, bias_t, gamma_t, beta_t, Ws0, Ws1, bias_s, gamma_s, beta_s, edge_index_t, edge_index_s):
    raise NotImplementedError("write your pallas kernel here")


# speedup vs baseline: 2.7063x; 2.7063x over previous
import functools

import jax
import jax.numpy as jnp
from jax import lax
from jax.experimental import pallas as pl
from jax.experimental.pallas import tpu as pltpu
from jax.experimental.pallas import tpu_sc as plsc

N = 50000
E = 800000
F = 32
EPS = 1e-5
SLOPE = 0.1

NCORE = 2
NSUB = 16
CH = 512
SB = 128
EP = 851968
NP = 50048
ROWS_PER_SUB = NP // NSUB


def _sc_spmm(table, src2, dst_r, w_p, split_cols):
  nchunks = (EP // NSUB) // CH if split_cols else (EP // (NCORE * NSUB)) // CH
  mesh = plsc.VectorSubcoreMesh(core_axis_name="c", subcore_axis_name="s")
  zeros = jnp.zeros((NP, F), jnp.float32)

  @functools.partial(
      pl.kernel,
      out_type=jax.ShapeDtypeStruct((NCORE, NP, F), jnp.float32),
      mesh=mesh,
      compiler_params=pltpu.CompilerParams(use_tc_tiling_on_sc=False),
      scratch_types=[
          pltpu.VMEM((CH,), jnp.int32),
          pltpu.VMEM((CH // SB, SB), jnp.int32),
          pltpu.VMEM((CH,), jnp.float32),
          pltpu.VMEM((CH, F), jnp.float32),
          pltpu.VMEM_SHARED((NP, F), jnp.float32),
          pltpu.SemaphoreType.DMA,
      ],
  )
  def spmm(table_h, src_h, dstr_h, w_h, zeros_h, out_h,
           srcv, dstv, wv, rows, acc, sem):
    c = lax.axis_index("c")
    s = lax.axis_index("s")

    r0 = pl.multiple_of(s * ROWS_PER_SUB, 8)
    pltpu.sync_copy(zeros_h.at[pl.ds(r0, ROWS_PER_SUB)],
                    acc.at[pl.ds(r0, ROWS_PER_SUB)])
    plsc.subcore_barrier()

    if split_cols:
      base = s * (EP // NSUB)
    else:
      base = (c * NSUB + s) * (EP // (NCORE * NSUB))

    def chunk_body(i, _):
      e0 = pl.multiple_of(base + i * CH, CH)
      pltpu.sync_copy(src_h.at[pl.ds(pl.multiple_of(c * EP + e0, CH), CH)], srcv)
      pltpu.sync_copy(w_h.at[pl.ds(e0, CH)], wv)
      pltpu.sync_copy(dstr_h.at[pl.ds(pl.multiple_of(e0 // SB, CH // SB), CH // SB)], dstv)
      pltpu.async_copy(table_h.at[srcv], rows, sem).wait()

      def group_body(g, _):
        w16 = wv[pl.ds(g * 16, 16)]
        for u in range(16):
          e = g * 16 + u
          ws = w16[u]
          for h in range(0, F, 16):
            rows[e, pl.ds(h, 16)] = rows[e, pl.ds(h, 16)] * ws
        return 0

      lax.fori_loop(0, CH // 16, group_body, 0)

      for j in range(CH // SB):
        pltpu.sync_copy(rows.at[pl.ds(j * SB, SB)],
                        acc.at[dstv.at[j]], add=True)
      return 0

    lax.fori_loop(0, nchunks, chunk_body, 0)

    plsc.subcore_barrier()
    pltpu.sync_copy(acc.at[pl.ds(r0, ROWS_PER_SUB)],
                    out_h.at[c, pl.ds(r0, ROWS_PER_SUB)])

  return spmm(table, src2, dst_r, w_p, zeros)


BN = 2000


def _tc_mm_body(x_ref, s2_ref, a_ref, b2_ref, bias_ref, y_ref, s1_ref, sq_ref):
  y = jnp.dot(x_ref[...], a_ref[...], preferred_element_type=jnp.float32)
  y -= jnp.dot(s2_ref[0], b2_ref[0], preferred_element_type=jnp.float32)
  y -= jnp.dot(s2_ref[1], b2_ref[1], preferred_element_type=jnp.float32)
  y += bias_ref[...]
  y_ref[...] = y
  part = jnp.sum(y.reshape(BN // 8, 8, F), axis=0)
  psq = jnp.sum((y * y).reshape(BN // 8, 8, F), axis=0)

  @pl.when(pl.program_id(0) == 0)
  def _():
    s1_ref[...] = part
    sq_ref[...] = psq

  @pl.when(pl.program_id(0) != 0)
  def _():
    s1_ref[...] += part
    sq_ref[...] += psq


def _tc_matmul(x, s2, a, b2, bias):
  d = x.shape[1]
  return pl.pallas_call(
      _tc_mm_body,
      grid=(N // BN,),
      in_specs=[
          pl.BlockSpec((BN, d), lambda i: (i, 0)),
          pl.BlockSpec((2, BN, F), lambda i: (0, i, 0)),
          pl.BlockSpec((d, F), lambda i: (0, 0)),
          pl.BlockSpec((2, F, F), lambda i: (0, 0, 0)),
          pl.BlockSpec((1, F), lambda i: (0, 0)),
      ],
      out_specs=[
          pl.BlockSpec((BN, F), lambda i: (i, 0)),
          pl.BlockSpec((8, F), lambda i: (0, 0)),
          pl.BlockSpec((8, F), lambda i: (0, 0)),
      ],
      out_shape=[
          jax.ShapeDtypeStruct((N, F), jnp.float32),
          jax.ShapeDtypeStruct((8, F), jnp.float32),
          jax.ShapeDtypeStruct((8, F), jnp.float32),
      ],
  )(x, s2, a, b2, bias)


def _tc_norm_body(y_ref, s1_ref, sq_ref, g_ref, b_ref, o_ref):
  s1 = jnp.sum(s1_ref[...], axis=0, keepdims=True)
  sq = jnp.sum(sq_ref[...], axis=0, keepdims=True)
  mean = s1 / N
  var = sq / N - mean * mean
  scale = g_ref[...] * lax.rsqrt(var + EPS)
  shift = b_ref[...] - mean * scale
  o = y_ref[...] * scale + shift
  o_ref[...] = jnp.where(o >= 0, o, SLOPE * o)


def _tc_norm(y, s1, sq, gamma, beta):
  return pl.pallas_call(
      _tc_norm_body,
      grid=(N // BN,),
      in_specs=[
          pl.BlockSpec((BN, F), lambda i: (i, 0)),
          pl.BlockSpec((8, F), lambda i: (0, 0)),
          pl.BlockSpec((8, F), lambda i: (0, 0)),
          pl.BlockSpec((1, F), lambda i: (0, 0)),
          pl.BlockSpec((1, F), lambda i: (0, 0)),
      ],
      out_specs=pl.BlockSpec((BN, F), lambda i: (i, 0)),
      out_shape=jax.ShapeDtypeStruct((N, F), jnp.float32),
  )(y, s1, sq, gamma, beta)


def _combine_weights(Ws):
  ks = jnp.arange(Ws.shape[0], dtype=jnp.float32)
  return jnp.sum(Ws, axis=0), jnp.einsum("k,kij->ij", ks, Ws)


def _conv_block(x, table, src2, dst_r, w_p, Ws, bias, gamma, beta, split_cols):
  a, b = _combine_weights(Ws)
  if split_cols:
    b2 = jnp.stack([b[:F], b[F:]])
  else:
    b2 = jnp.stack([b, b])
  s2 = _sc_spmm(table, src2, dst_r, w_p, split_cols)[:, :N, :]
  y, s1, sq = _tc_matmul(x, s2, a, b2, bias.reshape(1, F))
  return _tc_norm(y, s1, sq, gamma.reshape(1, F), beta.reshape(1, F))


def _prep_edges(ei, w):
  pad = EP - E
  src = jnp.concatenate([ei[0].astype(jnp.int32), jnp.zeros((pad,), jnp.int32)])
  dst = jnp.concatenate([ei[1].astype(jnp.int32), jnp.zeros((pad,), jnp.int32)])
  w_p = jnp.concatenate([w, jnp.zeros((pad,), jnp.float32)])
  src2_off = jnp.concatenate([src, src + N])
  src2_eq = jnp.concatenate([src, src])
  return src2_off, src2_eq, dst.reshape(EP // SB, SB), w_p


def _stream(x0, ei, w, W0, W1, bias, gamma, beta):
  src2_off, src2_eq, dst_r, w_p = _prep_edges(ei, w)
  table0 = jnp.concatenate([x0[:, :F], x0[:, F:]], axis=0)
  h = _conv_block(x0, table0, src2_off, dst_r, w_p,
                  W0, bias[0], gamma[0], beta[0], split_cols=True)
  return _conv_block(h, h, src2_eq, dst_r, w_p,
                     W1, bias[1], gamma[1], beta[1], split_cols=False)


def kernel(x_t0, edge_weight_t, x_s0, edge_weight_s, Wt0, Wt1, bias_t, gamma_t,
           beta_t, Ws0, Ws1, bias_s, gamma_s, beta_s, edge_index_t,
           edge_index_s):
  out_t = _stream(x_t0, edge_index_t, edge_weight_t,
                  Wt0, Wt1, bias_t, gamma_t, beta_t)
  out_s = _stream(x_s0, edge_index_s, edge_weight_s,
                  Ws0, Ws1, bias_s, gamma_s, beta_s)
  return (out_t, out_s)


# speedup vs baseline: 7.0028x; 2.5876x over previous
import functools

import jax
import jax.numpy as jnp
from jax import lax
from jax.experimental import pallas as pl
from jax.experimental.pallas import tpu as pltpu
from jax.experimental.pallas import tpu_sc as plsc

N = 50000
E = 800000
F = 32
EPS = 1e-5
SLOPE = 0.1

NCORE = 2
NSUB = 16
CH = 256
SB = 128
NBUF = 3
EP = 811008
NP = 50048
ROWS_PER_SUB = NP // NSUB


def _sc_spmm(table, src2, dst_r, w_p, split_cols):
  nchunks = (EP // NSUB) // CH if split_cols else (EP // (NCORE * NSUB)) // CH
  ntrip = nchunks // NBUF
  mesh = plsc.VectorSubcoreMesh(core_axis_name="c", subcore_axis_name="s")
  zeros = jnp.zeros((NP, F), jnp.float32)
  NSC = CH // SB

  @functools.partial(
      pl.kernel,
      out_type=jax.ShapeDtypeStruct((NCORE, NP, F), jnp.float32),
      mesh=mesh,
      compiler_params=pltpu.CompilerParams(use_tc_tiling_on_sc=False),
      scratch_types=[
          pltpu.VMEM((NBUF, CH), jnp.int32),
          pltpu.VMEM((NBUF, NSC, SB), jnp.int32),
          pltpu.VMEM((NBUF, CH), jnp.float32),
          pltpu.VMEM((NBUF, CH, F), jnp.float32),
          pltpu.VMEM_SHARED((NP, F), jnp.float32),
          pltpu.SemaphoreType.DMA((NBUF,)),
          pltpu.SemaphoreType.DMA((NBUF,)),
          pltpu.SemaphoreType.DMA((NBUF,)),
      ],
  )
  def spmm(table_h, src_h, dstr_h, w_h, zeros_h, out_h,
           srcv, dstv, wv, rows, acc, gsem, ssem, lsem):
    c = lax.axis_index("c")
    s = lax.axis_index("s")

    r0 = pl.multiple_of(s * ROWS_PER_SUB, 8)
    pltpu.sync_copy(zeros_h.at[pl.ds(r0, ROWS_PER_SUB)],
                    acc.at[pl.ds(r0, ROWS_PER_SUB)])
    plsc.subcore_barrier()

    if split_cols:
      base = s * (EP // NSUB)
    else:
      base = (c * NSUB + s) * (EP // (NCORE * NSUB))

    def load_linear(k, b):
      e0 = pl.multiple_of(base + k * CH, CH)
      d1 = pltpu.async_copy(
          src_h.at[pl.ds(pl.multiple_of(c * EP + e0, CH), CH)],
          srcv.at[b], lsem.at[b])
      d2 = pltpu.async_copy(w_h.at[pl.ds(e0, CH)], wv.at[b], lsem.at[b])
      d3 = pltpu.async_copy(
          dstr_h.at[pl.ds(pl.multiple_of(e0 // SB, NSC), NSC)],
          dstv.at[b], lsem.at[b])
      return d1, d2, d3

    def issue_gather(b):
      pltpu.async_copy(table_h.at[srcv.at[b]], rows.at[b], gsem.at[b])

    def wait_gather(b):
      pltpu.make_async_copy(zeros_h.at[pl.ds(0, CH)], rows.at[b],
                            gsem.at[b]).wait()

    def multiply(b):
      def group_body(g, _):
        w16 = wv[b, pl.ds(g * 16, 16)]
        for u in range(16):
          e = g * 16 + u
          ws = w16[u]
          for h in range(0, F, 16):
            rows[b, e, pl.ds(h, 16)] = rows[b, e, pl.ds(h, 16)] * ws
        return 0
      lax.fori_loop(0, CH // 16, group_body, 0)

    def issue_scatters(b):
      for j in range(NSC):
        pltpu.async_copy(rows.at[b, pl.ds(j * SB, SB)],
                         acc.at[dstv.at[b, j]], ssem.at[b], add=True)

    def drain_scatters(b):
      for j in range(NSC):
        pltpu.make_async_copy(zeros_h.at[pl.ds(0, SB)],
                              rows.at[b, pl.ds(j * SB, SB)], ssem.at[b]).wait()

    for b in range(NBUF - 1):
      for d in load_linear(b, b):
        d.wait()
      issue_gather(b)

    def trip_body(t, _):
      for b in range(NBUF):
        k = t * NBUF + b
        bp = (b + NBUF - 1) % NBUF
        wait_gather(b)
        multiply(b)
        issue_scatters(b)

        def refill(j):
          drain_scatters(bp)
          d1, d2, d3 = load_linear(j, bp)
          d1.wait(); d2.wait(); d3.wait()
          issue_gather(bp)

        if b == 0:
          @pl.when(t == 0)
          def _():
            d1, d2, d3 = load_linear(NBUF - 1, bp)
            d1.wait(); d2.wait(); d3.wait()
            issue_gather(bp)

          @pl.when(t > 0)
          def _():
            refill(k + NBUF - 1)
        elif b == 1:
          @pl.when(t < ntrip - 1)
          def _():
            refill(k + NBUF - 1)

          @pl.when(t == ntrip - 1)
          def _():
            drain_scatters(bp)
        else:
          @pl.when(t < ntrip - 1)
          def _():
            refill(k + NBUF - 1)

          @pl.when(t == ntrip - 1)
          def _():
            drain_scatters(bp)
      return 0

    lax.fori_loop(0, ntrip, trip_body, 0)

    drain_scatters((nchunks - 1) % NBUF)

    plsc.subcore_barrier()
    pltpu.sync_copy(acc.at[pl.ds(r0, ROWS_PER_SUB)],
                    out_h.at[c, pl.ds(r0, ROWS_PER_SUB)])

  return spmm(table, src2, dst_r, w_p, zeros)


BN = 2000


def _tc_mm_body(x_ref, s2_ref, a_ref, b2_ref, bias_ref, y_ref, s1_ref, sq_ref):
  y = jnp.dot(x_ref[...], a_ref[...], preferred_element_type=jnp.float32)
  y -= jnp.dot(s2_ref[0], b2_ref[0], preferred_element_type=jnp.float32)
  y -= jnp.dot(s2_ref[1], b2_ref[1], preferred_element_type=jnp.float32)
  y += bias_ref[...]
  y_ref[...] = y
  part = jnp.sum(y.reshape(BN // 8, 8, F), axis=0)
  psq = jnp.sum((y * y).reshape(BN // 8, 8, F), axis=0)

  @pl.when(pl.program_id(0) == 0)
  def _():
    s1_ref[...] = part
    sq_ref[...] = psq

  @pl.when(pl.program_id(0) != 0)
  def _():
    s1_ref[...] += part
    sq_ref[...] += psq


def _tc_matmul(x, s2, a, b2, bias):
  d = x.shape[1]
  return pl.pallas_call(
      _tc_mm_body,
      grid=(N // BN,),
      in_specs=[
          pl.BlockSpec((BN, d), lambda i: (i, 0)),
          pl.BlockSpec((2, BN, F), lambda i: (0, i, 0)),
          pl.BlockSpec((d, F), lambda i: (0, 0)),
          pl.BlockSpec((2, F, F), lambda i: (0, 0, 0)),
          pl.BlockSpec((1, F), lambda i: (0, 0)),
      ],
      out_specs=[
          pl.BlockSpec((BN, F), lambda i: (i, 0)),
          pl.BlockSpec((8, F), lambda i: (0, 0)),
          pl.BlockSpec((8, F), lambda i: (0, 0)),
      ],
      out_shape=[
          jax.ShapeDtypeStruct((N, F), jnp.float32),
          jax.ShapeDtypeStruct((8, F), jnp.float32),
          jax.ShapeDtypeStruct((8, F), jnp.float32),
      ],
  )(x, s2, a, b2, bias)


def _tc_norm_body(y_ref, s1_ref, sq_ref, g_ref, b_ref, o_ref):
  s1 = jnp.sum(s1_ref[...], axis=0, keepdims=True)
  sq = jnp.sum(sq_ref[...], axis=0, keepdims=True)
  mean = s1 / N
  var = sq / N - mean * mean
  scale = g_ref[...] * lax.rsqrt(var + EPS)
  shift = b_ref[...] - mean * scale
  o = y_ref[...] * scale + shift
  o_ref[...] = jnp.where(o >= 0, o, SLOPE * o)


def _tc_norm(y, s1, sq, gamma, beta):
  return pl.pallas_call(
      _tc_norm_body,
      grid=(N // BN,),
      in_specs=[
          pl.BlockSpec((BN, F), lambda i: (i, 0)),
          pl.BlockSpec((8, F), lambda i: (0, 0)),
          pl.BlockSpec((8, F), lambda i: (0, 0)),
          pl.BlockSpec((1, F), lambda i: (0, 0)),
          pl.BlockSpec((1, F), lambda i: (0, 0)),
      ],
      out_specs=pl.BlockSpec((BN, F), lambda i: (i, 0)),
      out_shape=jax.ShapeDtypeStruct((N, F), jnp.float32),
  )(y, s1, sq, gamma, beta)


def _combine_weights(Ws):
  ks = jnp.arange(Ws.shape[0], dtype=jnp.float32)
  return jnp.sum(Ws, axis=0), jnp.einsum("k,kij->ij", ks, Ws)


def _conv_block(x, table, src2, dst_r, w_p, Ws, bias, gamma, beta, split_cols):
  a, b = _combine_weights(Ws)
  if split_cols:
    b2 = jnp.stack([b[:F], b[F:]])
  else:
    b2 = jnp.stack([b, b])
  s2 = _sc_spmm(table, src2, dst_r, w_p, split_cols)[:, :N, :]
  y, s1, sq = _tc_matmul(x, s2, a, b2, bias.reshape(1, F))
  return _tc_norm(y, s1, sq, gamma.reshape(1, F), beta.reshape(1, F))


def _prep_edges(ei, w):
  pad = EP - E
  src = jnp.concatenate([ei[0].astype(jnp.int32), jnp.zeros((pad,), jnp.int32)])
  dst = jnp.concatenate([ei[1].astype(jnp.int32), jnp.zeros((pad,), jnp.int32)])
  w_p = jnp.concatenate([w, jnp.zeros((pad,), jnp.float32)])
  src2_off = jnp.concatenate([src, src + N])
  src2_eq = jnp.concatenate([src, src])
  return src2_off, src2_eq, dst.reshape(EP // SB, SB), w_p


def _stream(x0, ei, w, W0, W1, bias, gamma, beta):
  src2_off, src2_eq, dst_r, w_p = _prep_edges(ei, w)
  table0 = jnp.concatenate([x0[:, :F], x0[:, F:]], axis=0)
  h = _conv_block(x0, table0, src2_off, dst_r, w_p,
                  W0, bias[0], gamma[0], beta[0], split_cols=True)
  return _conv_block(h, h, src2_eq, dst_r, w_p,
                     W1, bias[1], gamma[1], beta[1], split_cols=False)


def kernel(x_t0, edge_weight_t, x_s0, edge_weight_s, Wt0, Wt1, bias_t, gamma_t,
           beta_t, Ws0, Ws1, bias_s, gamma_s, beta_s, edge_index_t,
           edge_index_s):
  out_t = _stream(x_t0, edge_index_t, edge_weight_t,
                  Wt0, Wt1, bias_t, gamma_t, beta_t)
  out_s = _stream(x_s0, edge_index_s, edge_weight_s,
                  Ws0, Ws1, bias_s, gamma_s, beta_s)
  return (out_t, out_s)
